# baseline (speedup 1.0000x reference)
import functools

import jax
import jax.numpy as jnp
import numpy as np
from jax import lax
from jax.experimental import pallas as pl
from jax.experimental.pallas import tpu as pltpu
from jax.experimental.pallas import tpu_sc as plsc

_VOCAB = 100000
_D = 64
_SEQ = 200
_BATCH = 4096

_NC = 2
_NS = 16
_NW = _NC * _NS
_ROWS = _BATCH * _SEQ
_RPW = _ROWS // _NW
_C = 2 * _SEQ
_NCHUNK = _RPW // _C
_G = 80
_NG = _C // _G
_CP = _C // 2
_QP = _SEQ // 2


def _positional_encoding() -> jnp.ndarray:
    pos = np.arange(_SEQ, dtype=np.float64)[:, None]
    div = np.exp(np.arange(0, _D, 2, dtype=np.float64) * (-np.log(10000.0) / _D))
    pe = np.zeros((_SEQ, _D), dtype=np.float32)
    pe[:, 0::2] = np.sin(pos * div).astype(np.float32)
    pe[:, 1::2] = np.cos(pos * div).astype(np.float32)
    return jnp.asarray(pe.reshape(_QP, 2 * _D))


_MESH = plsc.VectorSubcoreMesh(core_axis_name="c", subcore_axis_name="s")


@functools.partial(
    pl.kernel,
    mesh=_MESH,
    out_type=jax.ShapeDtypeStruct((_ROWS // 2, 2 * _D), jnp.float32),
    scratch_types=[
        pltpu.VMEM((_C,), jnp.int32),
        pltpu.VMEM((_C,), jnp.int32),
        pltpu.VMEM((_C, _D), jnp.float32),
        pltpu.VMEM((_C, _D), jnp.float32),
        pltpu.VMEM((_CP, 2 * _D), jnp.float32),
        pltpu.VMEM((_CP, 2 * _D), jnp.float32),
        pltpu.VMEM((_QP, 2 * _D), jnp.float32),
        pltpu.SemaphoreType.DMA,
        pltpu.SemaphoreType.DMA,
        pltpu.SemaphoreType.DMA,
        pltpu.SemaphoreType.DMA,
    ],
    compiler_params=pltpu.CompilerParams(use_tc_tiling_on_sc=False),
)
def _embed(idx_hbm, table_hbm, pe_hbm, out_hbm,
           idx0, idx1, g0, g1, o0, o1, pe_v, sg0, sg1, so0, so1):
    wid = lax.axis_index("s") * _NC + lax.axis_index("c")
    base = wid * _RPW
    base2 = base // 2
    pltpu.sync_copy(pe_hbm, pe_v)
    slots = ((idx0, g0, o0, sg0, so0), (idx1, g1, o1, sg1, so1))

    def fire_gathers(idx_b, g_b, sem):
        for g in range(_NG):
            pltpu.async_copy(
                table_hbm.at[idx_b.at[pl.ds(g * _G, _G)]],
                g_b.at[pl.ds(g * _G, _G)],
                sem,
            )

    def wait_gathers(idx_b, g_b, sem):
        for g in range(_NG):
            pltpu.make_async_copy(
                table_hbm.at[idx_b.at[pl.ds(g * _G, _G)]],
                g_b.at[pl.ds(g * _G, _G)],
                sem,
            ).wait()

    def wait_out(o_b, sem):
        pltpu.make_async_copy(o_b, out_hbm.at[pl.ds(base2, _CP)], sem).wait()

    pltpu.sync_copy(idx_hbm.at[pl.ds(base, _C)], idx0)
    pltpu.sync_copy(idx_hbm.at[pl.ds(base + _C, _C)], idx1)
    fire_gathers(idx0, g0, sg0)

    def body(ci, carry):
        for b in range(2):
            idx_b, g_b, o_b, sg_b, so_b = slots[b]
            idx_n, g_n, o_n, sg_n, so_n = slots[1 - b]
            cc = ci * 2 + b
            off = base + cc * _C
            wait_gathers(idx_b, g_b, sg_b)

            def pack(q, _):
                for h in range(2):
                    orow = h * _QP + q
                    for j in range(8):
                        grow = h * _SEQ + 2 * q + (j // 4)
                        src = pl.ds((j % 4) * 16, 16)
                        dst = pl.ds(j * 16, 16)
                        o_b[orow, dst] = g_b[grow, src] + pe_v[q, dst]
                return _

            lax.fori_loop(0, _QP, pack, 0)
            pltpu.async_copy(o_b, out_hbm.at[pl.ds(base2 + cc * _CP, _CP)], so_b)

            @pl.when(cc + 2 < _NCHUNK)
            def _():
                pltpu.sync_copy(idx_hbm.at[pl.ds(off + 2 * _C, _C)], idx_b)

            @pl.when(cc + 1 < _NCHUNK)
            def _():
                @pl.when(cc >= 1)
                def _():
                    wait_out(o_n, so_n)

                fire_gathers(idx_n, g_n, sg_n)

        return carry

    lax.fori_loop(0, _NCHUNK // 2, body, 0)
    wait_out(o0, so0)
    wait_out(o1, so1)


def kernel(x, table):
    idx = x.reshape(-1).astype(jnp.int32)
    out = _embed(idx, table, _positional_encoding())
    return out.reshape(_BATCH, _SEQ, _D)
